# baseline (speedup 1.0000x reference)
import functools

import jax
import jax.numpy as jnp
from jax import lax
from jax.experimental import pallas as pl
from jax.experimental.pallas import tpu as pltpu
from jax.experimental.pallas import tpu_sc as plsc

_N = 10000
_E = 320000
_D = 128
_NC = 2
_NS = 16
_NW = _NC * _NS
_CHUNK = 128
_NPAD = 10240
_STRIPE = _NPAD // _NS

_mesh = plsc.VectorSubcoreMesh(core_axis_name="core", subcore_axis_name="subcore")


def _deg_body(dst2d_hbm, deg_hbm, idx_d, ones_v, zbuf, acc, dsem):
    c = lax.axis_index("core")
    s = lax.axis_index("subcore")
    wid = c * _NS + s

    pltpu.sync_copy(dst2d_hbm.at[pl.ds(wid * _CPT, _CPT)], idx_d)

    @pl.loop(0, _CHUNK // 16)
    def _(k):
        ones_v[pl.ds(k * 16, 16)] = jnp.ones((16,), jnp.float32)

    @pl.loop(0, _STRIPE // 16)
    def _(k):
        zbuf[pl.ds(k * 16, 16)] = jnp.zeros((16,), jnp.float32)

    pltpu.sync_copy(zbuf, acc.at[pl.ds(s * _STRIPE, _STRIPE)])
    plsc.subcore_barrier()

    @pl.loop(0, _CPT)
    def _(i):
        pltpu.async_copy(ones_v, acc.at[idx_d.at[i]], dsem, add=True)

        @pl.when(i >= 4)
        def _():
            pltpu.make_async_copy(ones_v, acc.at[pl.ds(0, _CHUNK)], dsem).wait()

    @pl.loop(0, 4)
    def _(i):
        pltpu.make_async_copy(ones_v, acc.at[pl.ds(0, _CHUNK)], dsem).wait()

    plsc.subcore_barrier()
    pltpu.sync_copy(acc.at[pl.ds(s * _STRIPE, _STRIPE)],
                    deg_hbm.at[pl.ds(c * _NPAD + s * _STRIPE, _STRIPE)])


@jax.jit
def _deg_call(dst2d):
    f = pl.kernel(
        _deg_body,
        out_type=jax.ShapeDtypeStruct((_NC * _NPAD,), jnp.float32),
        mesh=_mesh,
        scratch_types=[
            pltpu.VMEM((_CPT, _CHUNK), jnp.int32),
            pltpu.VMEM((_CHUNK,), jnp.float32),
            pltpu.VMEM((_STRIPE,), jnp.float32),
            pltpu.VMEM_SHARED((_NPAD,), jnp.float32),
            pltpu.SemaphoreType.DMA,
        ],
    )
    return f(dst2d)


_CPT = 80
_EROWS = _NW * _CPT


def _agg_body(hs_hbm, src_hbm, dst_hbm, out_hbm, idx_s, idx_d, rows0, rows1,
              acc, gsem0, gsem1, ssem0, ssem1):
    c = lax.axis_index("core")
    s = lax.axis_index("subcore")
    wid = c * _NS + s

    @pl.loop(0, _CHUNK)
    def _(r):
        @pl.loop(0, _D // 16)
        def _(k):
            rows0[r, pl.ds(k * 16, 16)] = jnp.zeros((16,), jnp.float32)

    @pl.loop(0, _STRIPE // _CHUNK)
    def _(j):
        pltpu.sync_copy(rows0, acc.at[pl.ds(s * _STRIPE + j * _CHUNK, _CHUNK)])
    plsc.subcore_barrier()

    bufs = ((rows0, gsem0, ssem0), (rows1, gsem1, ssem1))
    half = _CPT // 2

    for h in (0, 1):
        if h:
            for rows, _g, ssem in bufs:
                pltpu.make_async_copy(rows, acc.at[pl.ds(0, _CHUNK)], ssem).wait()
        r0 = wid * _CPT + h * half
        pltpu.sync_copy(src_hbm.at[pl.ds(r0, half)], idx_s)
        pltpu.sync_copy(dst_hbm.at[pl.ds(r0, half)], idx_d)

        @pl.loop(0, half // 2)
        def _(j):
            for b, (rows, gsem, ssem) in enumerate(bufs):
                i = 2 * j + b

                @pl.when(j > 0)
                def _():
                    pltpu.make_async_copy(rows, acc.at[pl.ds(0, _CHUNK)], ssem).wait()

                pltpu.async_copy(hs_hbm.at[idx_s.at[i]], rows, gsem).wait()
                pltpu.async_copy(rows, acc.at[idx_d.at[i]], ssem, add=True)

    for rows, _g, ssem in bufs:
        pltpu.make_async_copy(rows, acc.at[pl.ds(0, _CHUNK)], ssem).wait()

    plsc.subcore_barrier()

    @pl.loop(0, _STRIPE // _CHUNK)
    def _(j):
        r0 = s * _STRIPE + j * _CHUNK
        pltpu.sync_copy(acc.at[pl.ds(r0, _CHUNK)], out_hbm.at[c].at[pl.ds(r0, _CHUNK)])


@jax.jit
def _agg_call(hs, src2d, dst2d):
    f = pl.kernel(
        _agg_body,
        out_type=jax.ShapeDtypeStruct((_NC, _NPAD, _D), jnp.float32),
        mesh=_mesh,
        scratch_types=[
            pltpu.VMEM((_CPT // 2, _CHUNK), jnp.int32),
            pltpu.VMEM((_CPT // 2, _CHUNK), jnp.int32),
            pltpu.VMEM((_CHUNK, _D), jnp.float32),
            pltpu.VMEM((_CHUNK, _D), jnp.float32),
            pltpu.VMEM_SHARED((_NPAD, _D), jnp.float32),
            pltpu.SemaphoreType.DMA,
            pltpu.SemaphoreType.DMA,
            pltpu.SemaphoreType.DMA,
            pltpu.SemaphoreType.DMA,
        ],
    )
    return f(hs, src2d, dst2d)


_BN = 2000
_GRID = _N // _BN


def _row_spec():
    return pl.BlockSpec((_BN, _D), lambda i: (i, 0))


def _col_spec():
    return pl.BlockSpec((_BN, 1), lambda i: (i, 0))


def _full_spec():
    return pl.BlockSpec((_D, _D), lambda i: (0, 0))


def _vec_spec():
    return pl.BlockSpec((1, _D), lambda i: (0, 0))


def _stage1_body(x_ref, w_ref, da_ref, db_ref, h_ref, hs_ref, dinv_ref):
    deg = da_ref[...] + db_ref[...] + 1.0
    dinv = lax.rsqrt(deg)
    h = jnp.dot(x_ref[...], w_ref[...], preferred_element_type=jnp.float32)
    h_ref[...] = h
    hs_ref[...] = h * dinv
    dinv_ref[...] = dinv


@jax.jit
def _stage1(x, W1, da, db):
    return pl.pallas_call(
        _stage1_body,
        grid=(_GRID,),
        in_specs=[_row_spec(), _full_spec(), _col_spec(), _col_spec()],
        out_specs=[_row_spec(), _row_spec(), _col_spec()],
        out_shape=[
            jax.ShapeDtypeStruct((_N, _D), jnp.float32),
            jax.ShapeDtypeStruct((_N, _D), jnp.float32),
            jax.ShapeDtypeStruct((_N, 1), jnp.float32),
        ],
    )(x, W1, da, db)


def _layer_norm(agg, g, beta):
    mu = jnp.mean(agg, axis=-1, keepdims=True)
    var = jnp.mean((agg - mu) ** 2, axis=-1, keepdims=True)
    return (agg - mu) * lax.rsqrt(var + 1e-5) * g + beta


def _stage2_body(sa_ref, sb_ref, h1_ref, dinv_ref, g_ref, beta_ref, b_ref,
                 w_ref, h2_ref, hs2_ref):
    dinv = dinv_ref[...]
    agg = dinv * (sa_ref[...] + sb_ref[...]) + dinv * dinv * h1_ref[...] + b_ref[...]
    y = jnp.maximum(_layer_norm(agg, g_ref[...], beta_ref[...]), 0.0)
    h2 = jnp.dot(y, w_ref[...], preferred_element_type=jnp.float32)
    h2_ref[...] = h2
    hs2_ref[...] = h2 * dinv


@jax.jit
def _stage2(sa, sb, h1, dinv, g1, beta1, b1, W2):
    return pl.pallas_call(
        _stage2_body,
        grid=(_GRID,),
        in_specs=[_row_spec(), _row_spec(), _row_spec(), _col_spec(),
                  _vec_spec(), _vec_spec(), _vec_spec(), _full_spec()],
        out_specs=[_row_spec(), _row_spec()],
        out_shape=[
            jax.ShapeDtypeStruct((_N, _D), jnp.float32),
            jax.ShapeDtypeStruct((_N, _D), jnp.float32),
        ],
    )(sa, sb, h1, dinv, g1, beta1, b1, W2)


def _stage3_body(sa_ref, sb_ref, h2_ref, dinv_ref, g_ref, beta_ref, b_ref,
                 x_ref, o_ref):
    dinv = dinv_ref[...]
    agg = dinv * (sa_ref[...] + sb_ref[...]) + dinv * dinv * h2_ref[...] + b_ref[...]
    y = _layer_norm(agg, g_ref[...], beta_ref[...])
    o_ref[...] = jnp.maximum(y + x_ref[...], 0.0)


@jax.jit
def _stage3(sa, sb, h2, dinv, g2, beta2, b2, x):
    return pl.pallas_call(
        _stage3_body,
        grid=(_GRID,),
        in_specs=[_row_spec(), _row_spec(), _row_spec(), _col_spec(),
                  _vec_spec(), _vec_spec(), _vec_spec(), _row_spec()],
        out_specs=_row_spec(),
        out_shape=jax.ShapeDtypeStruct((_N, _D), jnp.float32),
    )(sa, sb, h2, dinv, g2, beta2, b2, x)


def kernel(x, edge_index, W1, b1, g1, beta1, W2, b2, g2, beta2):
    npad_e = _EROWS * _CHUNK - _E
    src_p = jnp.concatenate(
        [edge_index[0].astype(jnp.int32), jnp.zeros((npad_e,), jnp.int32)])
    dst_p = jnp.concatenate(
        [edge_index[1].astype(jnp.int32),
         _N + (jnp.arange(npad_e, dtype=jnp.int32) % (_NPAD - _N))])
    src2d = src_p.reshape(_EROWS, _CHUNK)
    dst2d = dst_p.reshape(_EROWS, _CHUNK)

    deg_p = _deg_call(dst2d).reshape(_NC, _NPAD)
    da = deg_p[0, :_N, None]
    db = deg_p[1, :_N, None]

    g1r, beta1r, b1r = g1[None, :], beta1[None, :], b1[None, :]
    g2r, beta2r, b2r = g2[None, :], beta2[None, :], b2[None, :]

    h1, hs1, dinv = _stage1(x, W1, da, db)
    s1 = _agg_call(hs1, src2d, dst2d)
    h2, hs2 = _stage2(s1[0, :_N], s1[1, :_N], h1, dinv, g1r, beta1r, b1r, W2)
    s2 = _agg_call(hs2, src2d, dst2d)
    return _stage3(s2[0, :_N], s2[1, :_N], h2, dinv, g2r, beta2r, b2r, x)


# speedup vs baseline: 1.2211x; 1.2211x over previous
import functools

import jax
import jax.numpy as jnp
import numpy as np
from jax import lax
from jax.experimental import pallas as pl
from jax.experimental.pallas import tpu as pltpu
from jax.experimental.pallas import tpu_sc as plsc

_N = 10000
_E = 320000
_D = 128
_NC = 2
_NS = 16
_NW = _NC * _NS
_CHUNK = 128
_NPAD = 10240
_STRIPE = _NPAD // _NS

_mesh = plsc.VectorSubcoreMesh(core_axis_name="core", subcore_axis_name="subcore")


def _deg_body(dst2d_hbm, deg_hbm, idx_d, ones_v, zbuf, acc, dsem):
    c = lax.axis_index("core")
    s = lax.axis_index("subcore")
    wid = c * _NS + s

    pltpu.sync_copy(dst2d_hbm.at[pl.ds(wid * _CPT, _CPT)], idx_d)

    @pl.loop(0, _CHUNK // 16)
    def _(k):
        ones_v[pl.ds(k * 16, 16)] = jnp.ones((16,), jnp.float32)

    @pl.loop(0, _STRIPE // 16)
    def _(k):
        zbuf[pl.ds(k * 16, 16)] = jnp.zeros((16,), jnp.float32)

    pltpu.sync_copy(zbuf, acc.at[pl.ds(s * _STRIPE, _STRIPE)])
    plsc.subcore_barrier()

    @pl.loop(0, _CPT)
    def _(i):
        pltpu.async_copy(ones_v, acc.at[idx_d.at[i]], dsem, add=True)

        @pl.when(i >= 4)
        def _():
            pltpu.make_async_copy(ones_v, acc.at[pl.ds(0, _CHUNK)], dsem).wait()

    @pl.loop(0, 4)
    def _(i):
        pltpu.make_async_copy(ones_v, acc.at[pl.ds(0, _CHUNK)], dsem).wait()

    plsc.subcore_barrier()
    pltpu.sync_copy(acc.at[pl.ds(s * _STRIPE, _STRIPE)],
                    deg_hbm.at[pl.ds(c * _NPAD + s * _STRIPE, _STRIPE)])


@jax.jit
def _deg_call(dst2d):
    f = pl.kernel(
        _deg_body,
        out_type=jax.ShapeDtypeStruct((_NC * _NPAD,), jnp.float32),
        mesh=_mesh,
        scratch_types=[
            pltpu.VMEM((_CPT, _CHUNK), jnp.int32),
            pltpu.VMEM((_CHUNK,), jnp.float32),
            pltpu.VMEM((_STRIPE,), jnp.float32),
            pltpu.VMEM_SHARED((_NPAD,), jnp.float32),
            pltpu.SemaphoreType.DMA,
        ],
    )
    return f(dst2d)


_CPT = 80
_EROWS = _NW * _CPT

def _make_perm():
    real = _E // _CHUNK
    order, ri, pi = [], 0, real
    for t in range(_NW):
        nreal = 79 if t < 4 else 78
        order.extend(range(ri, ri + nreal))
        ri += nreal
        order.extend(range(pi, pi + _CPT - nreal))
        pi += _CPT - nreal
    assert ri == real and pi == _EROWS
    return np.asarray(order, np.int32)

_PERM = _make_perm()


def _agg_body(hs_hbm, src_hbm, dst_hbm, out_hbm, idx_s, idx_d, rows0, rows1,
              acc, gsem0, gsem1, ssem0, ssem1):
    c = lax.axis_index("core")
    s = lax.axis_index("subcore")
    wid = c * _NS + s

    @pl.loop(0, _CHUNK)
    def _(r):
        @pl.loop(0, _D // 16)
        def _(k):
            rows0[r, pl.ds(k * 16, 16)] = jnp.zeros((16,), jnp.float32)

    @pl.loop(0, _STRIPE // _CHUNK)
    def _(j):
        pltpu.sync_copy(rows0, acc.at[pl.ds(s * _STRIPE + j * _CHUNK, _CHUNK)])
    plsc.subcore_barrier()

    bufs = ((rows0, gsem0, ssem0), (rows1, gsem1, ssem1))
    half = _CPT // 2

    for h in (0, 1):
        if h:
            for rows, _g, ssem in bufs:
                pltpu.make_async_copy(rows, acc.at[pl.ds(0, _CHUNK)], ssem).wait()
        r0 = wid * _CPT + h * half
        pltpu.sync_copy(src_hbm.at[pl.ds(r0, half)], idx_s)
        pltpu.sync_copy(dst_hbm.at[pl.ds(r0, half)], idx_d)

        @pl.loop(0, half // 2)
        def _(j):
            for b, (rows, gsem, ssem) in enumerate(bufs):
                i = 2 * j + b

                @pl.when(j > 0)
                def _():
                    pltpu.make_async_copy(rows, acc.at[pl.ds(0, _CHUNK)], ssem).wait()

                pltpu.async_copy(hs_hbm.at[idx_s.at[i]], rows, gsem).wait()
                pltpu.async_copy(rows, acc.at[idx_d.at[i]], ssem, add=True)

    for rows, _g, ssem in bufs:
        pltpu.make_async_copy(rows, acc.at[pl.ds(0, _CHUNK)], ssem).wait()

    plsc.subcore_barrier()

    @pl.loop(0, _STRIPE // _CHUNK)
    def _(j):
        r0 = s * _STRIPE + j * _CHUNK
        pltpu.sync_copy(acc.at[pl.ds(r0, _CHUNK)], out_hbm.at[c].at[pl.ds(r0, _CHUNK)])


@jax.jit
def _agg_call(hs, src2d, dst2d):
    f = pl.kernel(
        _agg_body,
        out_type=jax.ShapeDtypeStruct((_NC, _NPAD, _D), jnp.float32),
        mesh=_mesh,
        scratch_types=[
            pltpu.VMEM((_CPT // 2, _CHUNK), jnp.int32),
            pltpu.VMEM((_CPT // 2, _CHUNK), jnp.int32),
            pltpu.VMEM((_CHUNK, _D), jnp.float32),
            pltpu.VMEM((_CHUNK, _D), jnp.float32),
            pltpu.VMEM_SHARED((_NPAD, _D), jnp.float32),
            pltpu.SemaphoreType.DMA,
            pltpu.SemaphoreType.DMA,
            pltpu.SemaphoreType.DMA,
            pltpu.SemaphoreType.DMA,
        ],
    )
    return f(hs, src2d, dst2d)


_BN = 2000
_GRID = _N // _BN


def _row_spec():
    return pl.BlockSpec((_BN, _D), lambda i: (i, 0))


def _col_spec():
    return pl.BlockSpec((_BN, 1), lambda i: (i, 0))


def _full_spec():
    return pl.BlockSpec((_D, _D), lambda i: (0, 0))


def _vec_spec():
    return pl.BlockSpec((1, _D), lambda i: (0, 0))


def _stage1_body(x_ref, w_ref, da_ref, db_ref, h_ref, hs_ref, dinv_ref):
    deg = da_ref[...] + db_ref[...] + 1.0
    dinv = lax.rsqrt(deg)
    h = jnp.dot(x_ref[...], w_ref[...], preferred_element_type=jnp.float32)
    h_ref[...] = h
    hs_ref[...] = h * dinv
    dinv_ref[...] = dinv


@jax.jit
def _stage1(x, W1, da, db):
    return pl.pallas_call(
        _stage1_body,
        grid=(_GRID,),
        in_specs=[_row_spec(), _full_spec(), _col_spec(), _col_spec()],
        out_specs=[_row_spec(), _row_spec(), _col_spec()],
        out_shape=[
            jax.ShapeDtypeStruct((_N, _D), jnp.float32),
            jax.ShapeDtypeStruct((_N, _D), jnp.float32),
            jax.ShapeDtypeStruct((_N, 1), jnp.float32),
        ],
    )(x, W1, da, db)


def _layer_norm(agg, g, beta):
    mu = jnp.mean(agg, axis=-1, keepdims=True)
    var = jnp.mean((agg - mu) ** 2, axis=-1, keepdims=True)
    return (agg - mu) * lax.rsqrt(var + 1e-5) * g + beta


def _stage2_body(sa_ref, sb_ref, h1_ref, dinv_ref, g_ref, beta_ref, b_ref,
                 w_ref, h2_ref, hs2_ref):
    dinv = dinv_ref[...]
    agg = dinv * (sa_ref[...] + sb_ref[...]) + dinv * dinv * h1_ref[...] + b_ref[...]
    y = jnp.maximum(_layer_norm(agg, g_ref[...], beta_ref[...]), 0.0)
    h2 = jnp.dot(y, w_ref[...], preferred_element_type=jnp.float32)
    h2_ref[...] = h2
    hs2_ref[...] = h2 * dinv


@jax.jit
def _stage2(sa, sb, h1, dinv, g1, beta1, b1, W2):
    return pl.pallas_call(
        _stage2_body,
        grid=(_GRID,),
        in_specs=[_row_spec(), _row_spec(), _row_spec(), _col_spec(),
                  _vec_spec(), _vec_spec(), _vec_spec(), _full_spec()],
        out_specs=[_row_spec(), _row_spec()],
        out_shape=[
            jax.ShapeDtypeStruct((_N, _D), jnp.float32),
            jax.ShapeDtypeStruct((_N, _D), jnp.float32),
        ],
    )(sa, sb, h1, dinv, g1, beta1, b1, W2)


def _stage3_body(sa_ref, sb_ref, h2_ref, dinv_ref, g_ref, beta_ref, b_ref,
                 x_ref, o_ref):
    dinv = dinv_ref[...]
    agg = dinv * (sa_ref[...] + sb_ref[...]) + dinv * dinv * h2_ref[...] + b_ref[...]
    y = _layer_norm(agg, g_ref[...], beta_ref[...])
    o_ref[...] = jnp.maximum(y + x_ref[...], 0.0)


@jax.jit
def _stage3(sa, sb, h2, dinv, g2, beta2, b2, x):
    return pl.pallas_call(
        _stage3_body,
        grid=(_GRID,),
        in_specs=[_row_spec(), _row_spec(), _row_spec(), _col_spec(),
                  _vec_spec(), _vec_spec(), _vec_spec(), _row_spec()],
        out_specs=_row_spec(),
        out_shape=jax.ShapeDtypeStruct((_N, _D), jnp.float32),
    )(sa, sb, h2, dinv, g2, beta2, b2, x)


def kernel(x, edge_index, W1, b1, g1, beta1, W2, b2, g2, beta2):
    npad_e = _EROWS * _CHUNK - _E
    src_p = jnp.concatenate(
        [edge_index[0].astype(jnp.int32), jnp.zeros((npad_e,), jnp.int32)])
    dst_p = jnp.concatenate(
        [edge_index[1].astype(jnp.int32),
         _N + (jnp.arange(npad_e, dtype=jnp.int32) % (_NPAD - _N))])
    perm = jnp.asarray(_PERM)
    src2d = src_p.reshape(_EROWS, _CHUNK)[perm]
    dst2d = dst_p.reshape(_EROWS, _CHUNK)[perm]

    deg_p = _deg_call(dst2d).reshape(_NC, _NPAD)
    da = deg_p[0, :_N, None]
    db = deg_p[1, :_N, None]

    g1r, beta1r, b1r = g1[None, :], beta1[None, :], b1[None, :]
    g2r, beta2r, b2r = g2[None, :], beta2[None, :], b2[None, :]

    h1, hs1, dinv = _stage1(x, W1, da, db)
    s1 = _agg_call(hs1, src2d, dst2d)
    h2, hs2 = _stage2(s1[0, :_N], s1[1, :_N], h1, dinv, g1r, beta1r, b1r, W2)
    s2 = _agg_call(hs2, src2d, dst2d)
    return _stage3(s2[0, :_N], s2[1, :_N], h2, dinv, g2r, beta2r, b2r, x)


# speedup vs baseline: 2.5097x; 2.0553x over previous
import functools

import jax
import jax.numpy as jnp
import numpy as np
from jax import lax
from jax.experimental import pallas as pl
from jax.experimental.pallas import tpu as pltpu
from jax.experimental.pallas import tpu_sc as plsc

_N = 10000
_E = 320000
_D = 128
_NC = 2
_NS = 16
_NW = _NC * _NS
_CHUNK = 128
_NPAD = 10240
_STRIPE = _NPAD // _NS

_mesh = plsc.VectorSubcoreMesh(core_axis_name="core", subcore_axis_name="subcore")


def _deg_body(dst2d_hbm, deg_hbm, idx_d, ones_v, zbuf, acc):
    c = lax.axis_index("core")
    s = lax.axis_index("subcore")
    wid = c * _NS + s

    pltpu.sync_copy(dst2d_hbm.at[pl.ds(wid * _CPT, _CPT)], idx_d)

    @pl.loop(0, _CHUNK // 16)
    def _(k):
        ones_v[pl.ds(k * 16, 16)] = jnp.ones((16,), jnp.float32)

    @pl.loop(0, _STRIPE // 16)
    def _(k):
        zbuf[pl.ds(k * 16, 16)] = jnp.zeros((16,), jnp.float32)

    pltpu.sync_copy(zbuf, acc.at[pl.ds(s * _STRIPE, _STRIPE)])
    plsc.subcore_barrier()

    nchunks = jnp.where(wid == _NW - 1, _CPT_LAST, _CPT)

    @pl.loop(0, nchunks)
    def _(i):
        pltpu.sync_copy(ones_v, acc.at[idx_d.at[i]], add=True)

    plsc.subcore_barrier()
    pltpu.sync_copy(acc.at[pl.ds(s * _STRIPE, _STRIPE)],
                    deg_hbm.at[pl.ds(c * _NPAD + s * _STRIPE, _STRIPE)])


@jax.jit
def _deg_call(dst2d):
    f = pl.kernel(
        _deg_body,
        out_type=jax.ShapeDtypeStruct((_NC * _NPAD,), jnp.float32),
        mesh=_mesh,
        scratch_types=[
            pltpu.VMEM((_CPT, _CHUNK), jnp.int32),
            pltpu.VMEM((_CHUNK,), jnp.float32),
            pltpu.VMEM((_STRIPE,), jnp.float32),
            pltpu.VMEM_SHARED((_NPAD,), jnp.float32),
        ],
    )
    return f(dst2d)


_CPT = 80
_EROWS = _NW * _CPT

_RROWS = _E // _CHUNK
_CPT_LAST = _RROWS - (_NW - 1) * _CPT


def _agg_body(hs_hbm, src_hbm, dst_hbm, out_hbm, idx_s, idx_d, rows, acc):
    c = lax.axis_index("core")
    s = lax.axis_index("subcore")
    wid = c * _NS + s

    @pl.loop(0, _CHUNK)
    def _(r):
        @pl.loop(0, _D // 16)
        def _(k):
            rows[r, pl.ds(k * 16, 16)] = jnp.zeros((16,), jnp.float32)

    @pl.loop(0, _STRIPE // _CHUNK)
    def _(j):
        pltpu.sync_copy(rows, acc.at[pl.ds(s * _STRIPE + j * _CHUNK, _CHUNK)])
    plsc.subcore_barrier()

    half = _CPT // 2
    last = wid == _NW - 1

    for h in (0, 1):
        r0 = wid * _CPT + h * half
        pltpu.sync_copy(src_hbm.at[pl.ds(r0, half)], idx_s)
        pltpu.sync_copy(dst_hbm.at[pl.ds(r0, half)], idx_d)
        nch = jnp.where(last, _CPT_LAST * (1 - h), half)

        @pl.loop(0, nch)
        def _(i):
            pltpu.sync_copy(hs_hbm.at[idx_s.at[i]], rows)
            pltpu.sync_copy(rows, acc.at[idx_d.at[i]], add=True)

    plsc.subcore_barrier()

    @pl.loop(0, _STRIPE // _CHUNK)
    def _(j):
        r0 = s * _STRIPE + j * _CHUNK
        pltpu.sync_copy(acc.at[pl.ds(r0, _CHUNK)], out_hbm.at[c].at[pl.ds(r0, _CHUNK)])


@jax.jit
def _agg_call(hs, src2d, dst2d):
    f = pl.kernel(
        _agg_body,
        out_type=jax.ShapeDtypeStruct((_NC, _NPAD, _D), jnp.float32),
        mesh=_mesh,
        scratch_types=[
            pltpu.VMEM((_CPT // 2, _CHUNK), jnp.int32),
            pltpu.VMEM((_CPT // 2, _CHUNK), jnp.int32),
            pltpu.VMEM((_CHUNK, _D), jnp.float32),
            pltpu.VMEM_SHARED((_NPAD, _D), jnp.float32),
        ],
    )
    return f(hs, src2d, dst2d)


_BN = 2000
_GRID = _N // _BN


def _row_spec():
    return pl.BlockSpec((_BN, _D), lambda i: (i, 0))


def _col_spec():
    return pl.BlockSpec((_BN, 1), lambda i: (i, 0))


def _full_spec():
    return pl.BlockSpec((_D, _D), lambda i: (0, 0))


def _vec_spec():
    return pl.BlockSpec((1, _D), lambda i: (0, 0))


def _stage1_body(x_ref, w_ref, da_ref, db_ref, h_ref, hs_ref, dinv_ref):
    deg = da_ref[...] + db_ref[...] + 1.0
    dinv = lax.rsqrt(deg)
    h = jnp.dot(x_ref[...], w_ref[...], preferred_element_type=jnp.float32)
    h_ref[...] = h
    hs_ref[...] = h * dinv
    dinv_ref[...] = dinv


@jax.jit
def _stage1(x, W1, da, db):
    return pl.pallas_call(
        _stage1_body,
        grid=(_GRID,),
        in_specs=[_row_spec(), _full_spec(), _col_spec(), _col_spec()],
        out_specs=[_row_spec(), _row_spec(), _col_spec()],
        out_shape=[
            jax.ShapeDtypeStruct((_N, _D), jnp.float32),
            jax.ShapeDtypeStruct((_N, _D), jnp.float32),
            jax.ShapeDtypeStruct((_N, 1), jnp.float32),
        ],
    )(x, W1, da, db)


def _layer_norm(agg, g, beta):
    mu = jnp.mean(agg, axis=-1, keepdims=True)
    var = jnp.mean((agg - mu) ** 2, axis=-1, keepdims=True)
    return (agg - mu) * lax.rsqrt(var + 1e-5) * g + beta


def _stage2_body(sa_ref, sb_ref, h1_ref, dinv_ref, g_ref, beta_ref, b_ref,
                 w_ref, h2_ref, hs2_ref):
    dinv = dinv_ref[...]
    agg = dinv * (sa_ref[...] + sb_ref[...]) + dinv * dinv * h1_ref[...] + b_ref[...]
    y = jnp.maximum(_layer_norm(agg, g_ref[...], beta_ref[...]), 0.0)
    h2 = jnp.dot(y, w_ref[...], preferred_element_type=jnp.float32)
    h2_ref[...] = h2
    hs2_ref[...] = h2 * dinv


@jax.jit
def _stage2(sa, sb, h1, dinv, g1, beta1, b1, W2):
    return pl.pallas_call(
        _stage2_body,
        grid=(_GRID,),
        in_specs=[_row_spec(), _row_spec(), _row_spec(), _col_spec(),
                  _vec_spec(), _vec_spec(), _vec_spec(), _full_spec()],
        out_specs=[_row_spec(), _row_spec()],
        out_shape=[
            jax.ShapeDtypeStruct((_N, _D), jnp.float32),
            jax.ShapeDtypeStruct((_N, _D), jnp.float32),
        ],
    )(sa, sb, h1, dinv, g1, beta1, b1, W2)


def _stage3_body(sa_ref, sb_ref, h2_ref, dinv_ref, g_ref, beta_ref, b_ref,
                 x_ref, o_ref):
    dinv = dinv_ref[...]
    agg = dinv * (sa_ref[...] + sb_ref[...]) + dinv * dinv * h2_ref[...] + b_ref[...]
    y = _layer_norm(agg, g_ref[...], beta_ref[...])
    o_ref[...] = jnp.maximum(y + x_ref[...], 0.0)


@jax.jit
def _stage3(sa, sb, h2, dinv, g2, beta2, b2, x):
    return pl.pallas_call(
        _stage3_body,
        grid=(_GRID,),
        in_specs=[_row_spec(), _row_spec(), _row_spec(), _col_spec(),
                  _vec_spec(), _vec_spec(), _vec_spec(), _row_spec()],
        out_specs=_row_spec(),
        out_shape=jax.ShapeDtypeStruct((_N, _D), jnp.float32),
    )(sa, sb, h2, dinv, g2, beta2, b2, x)


def kernel(x, edge_index, W1, b1, g1, beta1, W2, b2, g2, beta2):
    npad_e = _EROWS * _CHUNK - _E
    src_p = jnp.concatenate(
        [edge_index[0].astype(jnp.int32), jnp.zeros((npad_e,), jnp.int32)])
    dst_p = jnp.concatenate(
        [edge_index[1].astype(jnp.int32),
         _N + (jnp.arange(npad_e, dtype=jnp.int32) % (_NPAD - _N))])
    src2d = src_p.reshape(_EROWS, _CHUNK)
    dst2d = dst_p.reshape(_EROWS, _CHUNK)

    deg_p = _deg_call(dst2d).reshape(_NC, _NPAD)
    da = deg_p[0, :_N, None]
    db = deg_p[1, :_N, None]

    g1r, beta1r, b1r = g1[None, :], beta1[None, :], b1[None, :]
    g2r, beta2r, b2r = g2[None, :], beta2[None, :], b2[None, :]

    h1, hs1, dinv = _stage1(x, W1, da, db)
    s1 = _agg_call(hs1, src2d, dst2d)
    h2, hs2 = _stage2(s1[0, :_N], s1[1, :_N], h1, dinv, g1r, beta1r, b1r, W2)
    s2 = _agg_call(hs2, src2d, dst2d)
    return _stage3(s2[0, :_N], s2[1, :_N], h2, dinv, g2r, beta2r, b2r, x)


# speedup vs baseline: 3.1533x; 1.2564x over previous
import functools

import jax
import jax.numpy as jnp
import numpy as np
from jax import lax
from jax.experimental import pallas as pl
from jax.experimental.pallas import tpu as pltpu
from jax.experimental.pallas import tpu_sc as plsc

_N = 10000
_E = 320000
_D = 128
_NC = 2
_NS = 16
_NW = _NC * _NS
_CHUNK = 128
_NPAD = 10240
_STRIPE = _NPAD // _NS

_mesh = plsc.VectorSubcoreMesh(core_axis_name="core", subcore_axis_name="subcore")


def _deg_body(dst2d_hbm, deg_hbm, idx_d, ones_v, zbuf, acc):
    c = lax.axis_index("core")
    s = lax.axis_index("subcore")
    wid = c * _NS + s

    pltpu.sync_copy(dst2d_hbm.at[pl.ds(wid * _CPT, _CPT)], idx_d)

    @pl.loop(0, _CHUNK // 16)
    def _(k):
        ones_v[pl.ds(k * 16, 16)] = jnp.ones((16,), jnp.float32)

    @pl.loop(0, _STRIPE // 16)
    def _(k):
        zbuf[pl.ds(k * 16, 16)] = jnp.zeros((16,), jnp.float32)

    pltpu.sync_copy(zbuf, acc.at[pl.ds(s * _STRIPE, _STRIPE)])
    plsc.subcore_barrier()

    nchunks = jnp.where(wid == _NW - 1, _CPT_LAST, _CPT)

    @pl.loop(0, nchunks)
    def _(i):
        pltpu.sync_copy(ones_v, acc.at[idx_d.at[i]], add=True)

    plsc.subcore_barrier()
    pltpu.sync_copy(acc.at[pl.ds(s * _STRIPE, _STRIPE)],
                    deg_hbm.at[pl.ds(c * _NPAD + s * _STRIPE, _STRIPE)])


@jax.jit
def _deg_call(dst2d):
    f = pl.kernel(
        _deg_body,
        out_type=jax.ShapeDtypeStruct((_NC * _NPAD,), jnp.float32),
        mesh=_mesh,
        scratch_types=[
            pltpu.VMEM((_CPT, _CHUNK), jnp.int32),
            pltpu.VMEM((_CHUNK,), jnp.float32),
            pltpu.VMEM((_STRIPE,), jnp.float32),
            pltpu.VMEM_SHARED((_NPAD,), jnp.float32),
        ],
    )
    return f(dst2d)


_CPT = 80
_EROWS = _NW * _CPT

_RROWS = _E // _CHUNK
_CPT_LAST = _RROWS - (_NW - 1) * _CPT


def _agg_body(hs_hbm, src_hbm, dst_hbm, out_hbm, idx_s, idx_d, rows0, rows1,
              acc, gsem):
    c = lax.axis_index("core")
    s = lax.axis_index("subcore")
    wid = c * _NS + s

    @pl.loop(0, _CHUNK)
    def _(r):
        @pl.loop(0, _D // 16)
        def _(k):
            rows0[r, pl.ds(k * 16, 16)] = jnp.zeros((16,), jnp.float32)

    @pl.loop(0, _STRIPE // _CHUNK)
    def _(j):
        pltpu.sync_copy(rows0, acc.at[pl.ds(s * _STRIPE + j * _CHUNK, _CHUNK)])
    plsc.subcore_barrier()

    half = _CPT // 2
    last = wid == _NW - 1

    for h in (0, 1):
        r0 = wid * _CPT + h * half
        pltpu.sync_copy(src_hbm.at[pl.ds(r0, half)], idx_s)
        pltpu.sync_copy(dst_hbm.at[pl.ds(r0, half)], idx_d)
        nch = jnp.where(last, _CPT_LAST * (1 - h), half)

        @pl.when(nch > 0)
        def _():
            pltpu.async_copy(hs_hbm.at[idx_s.at[0]], rows0, gsem)

        @pl.loop(0, nch)
        def _(i):
            even = lax.rem(i, 2) == 0
            pltpu.make_async_copy(hs_hbm.at[pl.ds(0, _CHUNK)], rows0, gsem).wait()

            @pl.when(i + 1 < nch)
            def _():
                @pl.when(even)
                def _():
                    pltpu.async_copy(hs_hbm.at[idx_s.at[i + 1]], rows1, gsem)

                @pl.when(jnp.logical_not(even))
                def _():
                    pltpu.async_copy(hs_hbm.at[idx_s.at[i + 1]], rows0, gsem)

            @pl.when(even)
            def _():
                pltpu.sync_copy(rows0, acc.at[idx_d.at[i]], add=True)

            @pl.when(jnp.logical_not(even))
            def _():
                pltpu.sync_copy(rows1, acc.at[idx_d.at[i]], add=True)

    plsc.subcore_barrier()

    @pl.loop(0, _STRIPE // _CHUNK)
    def _(j):
        r0 = s * _STRIPE + j * _CHUNK
        pltpu.sync_copy(acc.at[pl.ds(r0, _CHUNK)], out_hbm.at[c].at[pl.ds(r0, _CHUNK)])


@jax.jit
def _agg_call(hs, src2d, dst2d):
    f = pl.kernel(
        _agg_body,
        out_type=jax.ShapeDtypeStruct((_NC, _NPAD, _D), jnp.float32),
        mesh=_mesh,
        scratch_types=[
            pltpu.VMEM((_CPT // 2, _CHUNK), jnp.int32),
            pltpu.VMEM((_CPT // 2, _CHUNK), jnp.int32),
            pltpu.VMEM((_CHUNK, _D), jnp.float32),
            pltpu.VMEM((_CHUNK, _D), jnp.float32),
            pltpu.VMEM_SHARED((_NPAD, _D), jnp.float32),
            pltpu.SemaphoreType.DMA,
        ],
    )
    return f(hs, src2d, dst2d)


_BN = 2000
_GRID = _N // _BN


def _row_spec():
    return pl.BlockSpec((_BN, _D), lambda i: (i, 0))


def _col_spec():
    return pl.BlockSpec((_BN, 1), lambda i: (i, 0))


def _full_spec():
    return pl.BlockSpec((_D, _D), lambda i: (0, 0))


def _vec_spec():
    return pl.BlockSpec((1, _D), lambda i: (0, 0))


def _stage1_body(x_ref, w_ref, da_ref, db_ref, h_ref, hs_ref, dinv_ref):
    deg = da_ref[...] + db_ref[...] + 1.0
    dinv = lax.rsqrt(deg)
    h = jnp.dot(x_ref[...], w_ref[...], preferred_element_type=jnp.float32)
    h_ref[...] = h
    hs_ref[...] = h * dinv
    dinv_ref[...] = dinv


@jax.jit
def _stage1(x, W1, da, db):
    return pl.pallas_call(
        _stage1_body,
        grid=(_GRID,),
        in_specs=[_row_spec(), _full_spec(), _col_spec(), _col_spec()],
        out_specs=[_row_spec(), _row_spec(), _col_spec()],
        out_shape=[
            jax.ShapeDtypeStruct((_N, _D), jnp.float32),
            jax.ShapeDtypeStruct((_N, _D), jnp.float32),
            jax.ShapeDtypeStruct((_N, 1), jnp.float32),
        ],
    )(x, W1, da, db)


def _layer_norm(agg, g, beta):
    mu = jnp.mean(agg, axis=-1, keepdims=True)
    var = jnp.mean((agg - mu) ** 2, axis=-1, keepdims=True)
    return (agg - mu) * lax.rsqrt(var + 1e-5) * g + beta


def _stage2_body(sa_ref, sb_ref, h1_ref, dinv_ref, g_ref, beta_ref, b_ref,
                 w_ref, h2_ref, hs2_ref):
    dinv = dinv_ref[...]
    agg = dinv * (sa_ref[...] + sb_ref[...]) + dinv * dinv * h1_ref[...] + b_ref[...]
    y = jnp.maximum(_layer_norm(agg, g_ref[...], beta_ref[...]), 0.0)
    h2 = jnp.dot(y, w_ref[...], preferred_element_type=jnp.float32)
    h2_ref[...] = h2
    hs2_ref[...] = h2 * dinv


@jax.jit
def _stage2(sa, sb, h1, dinv, g1, beta1, b1, W2):
    return pl.pallas_call(
        _stage2_body,
        grid=(_GRID,),
        in_specs=[_row_spec(), _row_spec(), _row_spec(), _col_spec(),
                  _vec_spec(), _vec_spec(), _vec_spec(), _full_spec()],
        out_specs=[_row_spec(), _row_spec()],
        out_shape=[
            jax.ShapeDtypeStruct((_N, _D), jnp.float32),
            jax.ShapeDtypeStruct((_N, _D), jnp.float32),
        ],
    )(sa, sb, h1, dinv, g1, beta1, b1, W2)


def _stage3_body(sa_ref, sb_ref, h2_ref, dinv_ref, g_ref, beta_ref, b_ref,
                 x_ref, o_ref):
    dinv = dinv_ref[...]
    agg = dinv * (sa_ref[...] + sb_ref[...]) + dinv * dinv * h2_ref[...] + b_ref[...]
    y = _layer_norm(agg, g_ref[...], beta_ref[...])
    o_ref[...] = jnp.maximum(y + x_ref[...], 0.0)


@jax.jit
def _stage3(sa, sb, h2, dinv, g2, beta2, b2, x):
    return pl.pallas_call(
        _stage3_body,
        grid=(_GRID,),
        in_specs=[_row_spec(), _row_spec(), _row_spec(), _col_spec(),
                  _vec_spec(), _vec_spec(), _vec_spec(), _row_spec()],
        out_specs=_row_spec(),
        out_shape=jax.ShapeDtypeStruct((_N, _D), jnp.float32),
    )(sa, sb, h2, dinv, g2, beta2, b2, x)


def kernel(x, edge_index, W1, b1, g1, beta1, W2, b2, g2, beta2):
    npad_e = _EROWS * _CHUNK - _E
    src_p = jnp.concatenate(
        [edge_index[0].astype(jnp.int32), jnp.zeros((npad_e,), jnp.int32)])
    dst_p = jnp.concatenate(
        [edge_index[1].astype(jnp.int32),
         _N + (jnp.arange(npad_e, dtype=jnp.int32) % (_NPAD - _N))])
    src2d = src_p.reshape(_EROWS, _CHUNK)
    dst2d = dst_p.reshape(_EROWS, _CHUNK)

    deg_p = _deg_call(dst2d).reshape(_NC, _NPAD)
    da = deg_p[0, :_N, None]
    db = deg_p[1, :_N, None]

    g1r, beta1r, b1r = g1[None, :], beta1[None, :], b1[None, :]
    g2r, beta2r, b2r = g2[None, :], beta2[None, :], b2[None, :]

    h1, hs1, dinv = _stage1(x, W1, da, db)
    s1 = _agg_call(hs1, src2d, dst2d)
    h2, hs2 = _stage2(s1[0, :_N], s1[1, :_N], h1, dinv, g1r, beta1r, b1r, W2)
    s2 = _agg_call(hs2, src2d, dst2d)
    return _stage3(s2[0, :_N], s2[1, :_N], h2, dinv, g2r, beta2r, b2r, x)


# speedup vs baseline: 3.5674x; 1.1313x over previous
import functools

import jax
import jax.numpy as jnp
import numpy as np
from jax import lax
from jax.experimental import pallas as pl
from jax.experimental.pallas import tpu as pltpu
from jax.experimental.pallas import tpu_sc as plsc

_N = 10000
_E = 320000
_D = 128
_NC = 2
_NS = 16
_NW = _NC * _NS
_CHUNK = 128
_NPAD = 10240
_STRIPE = _NPAD // _NS

_mesh = plsc.VectorSubcoreMesh(core_axis_name="core", subcore_axis_name="subcore")


def _deg_body(dst2d_hbm, deg_hbm, idx_d, ones_v, zbuf, acc):
    c = lax.axis_index("core")
    s = lax.axis_index("subcore")
    wid = c * _NS + s

    pltpu.sync_copy(dst2d_hbm.at[pl.ds(wid * _CPT, _CPT)], idx_d)

    @pl.loop(0, _CHUNK // 16)
    def _(k):
        ones_v[pl.ds(k * 16, 16)] = jnp.ones((16,), jnp.float32)

    @pl.loop(0, _STRIPE // 16)
    def _(k):
        zbuf[pl.ds(k * 16, 16)] = jnp.zeros((16,), jnp.float32)

    pltpu.sync_copy(zbuf, acc.at[pl.ds(s * _STRIPE, _STRIPE)])
    plsc.subcore_barrier()

    nchunks = jnp.where(wid == _NW - 1, _CPT_LAST, _CPT)

    @pl.loop(0, nchunks)
    def _(i):
        pltpu.sync_copy(ones_v, acc.at[idx_d.at[i]], add=True)

    plsc.subcore_barrier()
    pltpu.sync_copy(acc.at[pl.ds(s * _STRIPE, _STRIPE)],
                    deg_hbm.at[pl.ds(c * _NPAD + s * _STRIPE, _STRIPE)])


@jax.jit
def _deg_call(dst2d):
    f = pl.kernel(
        _deg_body,
        out_type=jax.ShapeDtypeStruct((_NC * _NPAD,), jnp.float32),
        mesh=_mesh,
        scratch_types=[
            pltpu.VMEM((_CPT, _CHUNK), jnp.int32),
            pltpu.VMEM((_CHUNK,), jnp.float32),
            pltpu.VMEM((_STRIPE,), jnp.float32),
            pltpu.VMEM_SHARED((_NPAD,), jnp.float32),
        ],
    )
    return f(dst2d)


_CPT = 80
_EROWS = _NW * _CPT

_RROWS = _E // _CHUNK
_CPT_LAST = _RROWS - (_NW - 1) * _CPT


def _agg_body(hs_hbm, src_hbm, dst_hbm, out_hbm, idx_s, idx_d, rows0, rows1,
              acc, gsem0, gsem1):
    c = lax.axis_index("core")
    s = lax.axis_index("subcore")
    wid = c * _NS + s

    @pl.loop(0, _CHUNK)
    def _(r):
        @pl.loop(0, _D // 16)
        def _(k):
            rows0[r, pl.ds(k * 16, 16)] = jnp.zeros((16,), jnp.float32)

    @pl.loop(0, _STRIPE // _CHUNK)
    def _(j):
        pltpu.sync_copy(rows0, acc.at[pl.ds(s * _STRIPE + j * _CHUNK, _CHUNK)])
    plsc.subcore_barrier()

    half = _CPT // 2
    last = wid == _NW - 1

    for h in (0, 1):
        r0 = wid * _CPT + h * half
        pltpu.sync_copy(src_hbm.at[pl.ds(r0, half)], idx_s)
        pltpu.sync_copy(dst_hbm.at[pl.ds(r0, half)], idx_d)
        nch = jnp.where(last, _CPT_LAST * (1 - h), half)

        @pl.when(nch > 0)
        def _():
            pltpu.async_copy(hs_hbm.at[idx_s.at[0]], rows0, gsem0)

        @pl.when(nch > 1)
        def _():
            pltpu.async_copy(hs_hbm.at[idx_s.at[1]], rows1, gsem1)

        @pl.loop(0, nch)
        def _(i):
            even = lax.rem(i, 2) == 0

            @pl.when(even)
            def _():
                pltpu.make_async_copy(hs_hbm.at[pl.ds(0, _CHUNK)], rows0, gsem0).wait()
                pltpu.sync_copy(rows0, acc.at[idx_d.at[i]], add=True)

                @pl.when(i + 2 < nch)
                def _():
                    pltpu.async_copy(hs_hbm.at[idx_s.at[i + 2]], rows0, gsem0)

            @pl.when(jnp.logical_not(even))
            def _():
                pltpu.make_async_copy(hs_hbm.at[pl.ds(0, _CHUNK)], rows1, gsem1).wait()
                pltpu.sync_copy(rows1, acc.at[idx_d.at[i]], add=True)

                @pl.when(i + 2 < nch)
                def _():
                    pltpu.async_copy(hs_hbm.at[idx_s.at[i + 2]], rows1, gsem1)

    plsc.subcore_barrier()

    @pl.loop(0, _STRIPE // _CHUNK)
    def _(j):
        r0 = s * _STRIPE + j * _CHUNK
        pltpu.sync_copy(acc.at[pl.ds(r0, _CHUNK)], out_hbm.at[c].at[pl.ds(r0, _CHUNK)])


@jax.jit
def _agg_call(hs, src2d, dst2d):
    f = pl.kernel(
        _agg_body,
        out_type=jax.ShapeDtypeStruct((_NC, _NPAD, _D), jnp.float32),
        mesh=_mesh,
        scratch_types=[
            pltpu.VMEM((_CPT // 2, _CHUNK), jnp.int32),
            pltpu.VMEM((_CPT // 2, _CHUNK), jnp.int32),
            pltpu.VMEM((_CHUNK, _D), jnp.float32),
            pltpu.VMEM((_CHUNK, _D), jnp.float32),
            pltpu.VMEM_SHARED((_NPAD, _D), jnp.float32),
            pltpu.SemaphoreType.DMA,
            pltpu.SemaphoreType.DMA,
        ],
    )
    return f(hs, src2d, dst2d)


_BN = 2000
_GRID = _N // _BN


def _row_spec():
    return pl.BlockSpec((_BN, _D), lambda i: (i, 0))


def _col_spec():
    return pl.BlockSpec((_BN, 1), lambda i: (i, 0))


def _full_spec():
    return pl.BlockSpec((_D, _D), lambda i: (0, 0))


def _vec_spec():
    return pl.BlockSpec((1, _D), lambda i: (0, 0))


def _stage1_body(x_ref, w_ref, da_ref, db_ref, h_ref, hs_ref, dinv_ref):
    deg = da_ref[...] + db_ref[...] + 1.0
    dinv = lax.rsqrt(deg)
    h = jnp.dot(x_ref[...], w_ref[...], preferred_element_type=jnp.float32)
    h_ref[...] = h
    hs_ref[...] = h * dinv
    dinv_ref[...] = dinv


@jax.jit
def _stage1(x, W1, da, db):
    return pl.pallas_call(
        _stage1_body,
        grid=(_GRID,),
        in_specs=[_row_spec(), _full_spec(), _col_spec(), _col_spec()],
        out_specs=[_row_spec(), _row_spec(), _col_spec()],
        out_shape=[
            jax.ShapeDtypeStruct((_N, _D), jnp.float32),
            jax.ShapeDtypeStruct((_N, _D), jnp.float32),
            jax.ShapeDtypeStruct((_N, 1), jnp.float32),
        ],
    )(x, W1, da, db)


def _layer_norm(agg, g, beta):
    mu = jnp.mean(agg, axis=-1, keepdims=True)
    var = jnp.mean((agg - mu) ** 2, axis=-1, keepdims=True)
    return (agg - mu) * lax.rsqrt(var + 1e-5) * g + beta


def _stage2_body(sa_ref, sb_ref, h1_ref, dinv_ref, g_ref, beta_ref, b_ref,
                 w_ref, h2_ref, hs2_ref):
    dinv = dinv_ref[...]
    agg = dinv * (sa_ref[...] + sb_ref[...]) + dinv * dinv * h1_ref[...] + b_ref[...]
    y = jnp.maximum(_layer_norm(agg, g_ref[...], beta_ref[...]), 0.0)
    h2 = jnp.dot(y, w_ref[...], preferred_element_type=jnp.float32)
    h2_ref[...] = h2
    hs2_ref[...] = h2 * dinv


@jax.jit
def _stage2(sa, sb, h1, dinv, g1, beta1, b1, W2):
    return pl.pallas_call(
        _stage2_body,
        grid=(_GRID,),
        in_specs=[_row_spec(), _row_spec(), _row_spec(), _col_spec(),
                  _vec_spec(), _vec_spec(), _vec_spec(), _full_spec()],
        out_specs=[_row_spec(), _row_spec()],
        out_shape=[
            jax.ShapeDtypeStruct((_N, _D), jnp.float32),
            jax.ShapeDtypeStruct((_N, _D), jnp.float32),
        ],
    )(sa, sb, h1, dinv, g1, beta1, b1, W2)


def _stage3_body(sa_ref, sb_ref, h2_ref, dinv_ref, g_ref, beta_ref, b_ref,
                 x_ref, o_ref):
    dinv = dinv_ref[...]
    agg = dinv * (sa_ref[...] + sb_ref[...]) + dinv * dinv * h2_ref[...] + b_ref[...]
    y = _layer_norm(agg, g_ref[...], beta_ref[...])
    o_ref[...] = jnp.maximum(y + x_ref[...], 0.0)


@jax.jit
def _stage3(sa, sb, h2, dinv, g2, beta2, b2, x):
    return pl.pallas_call(
        _stage3_body,
        grid=(_GRID,),
        in_specs=[_row_spec(), _row_spec(), _row_spec(), _col_spec(),
                  _vec_spec(), _vec_spec(), _vec_spec(), _row_spec()],
        out_specs=_row_spec(),
        out_shape=jax.ShapeDtypeStruct((_N, _D), jnp.float32),
    )(sa, sb, h2, dinv, g2, beta2, b2, x)


def kernel(x, edge_index, W1, b1, g1, beta1, W2, b2, g2, beta2):
    npad_e = _EROWS * _CHUNK - _E
    src_p = jnp.concatenate(
        [edge_index[0].astype(jnp.int32), jnp.zeros((npad_e,), jnp.int32)])
    dst_p = jnp.concatenate(
        [edge_index[1].astype(jnp.int32),
         _N + (jnp.arange(npad_e, dtype=jnp.int32) % (_NPAD - _N))])
    src2d = src_p.reshape(_EROWS, _CHUNK)
    dst2d = dst_p.reshape(_EROWS, _CHUNK)

    deg_p = _deg_call(dst2d).reshape(_NC, _NPAD)
    da = deg_p[0, :_N, None]
    db = deg_p[1, :_N, None]

    g1r, beta1r, b1r = g1[None, :], beta1[None, :], b1[None, :]
    g2r, beta2r, b2r = g2[None, :], beta2[None, :], b2[None, :]

    h1, hs1, dinv = _stage1(x, W1, da, db)
    s1 = _agg_call(hs1, src2d, dst2d)
    h2, hs2 = _stage2(s1[0, :_N], s1[1, :_N], h1, dinv, g1r, beta1r, b1r, W2)
    s2 = _agg_call(hs2, src2d, dst2d)
    return _stage3(s2[0, :_N], s2[1, :_N], h2, dinv, g2r, beta2r, b2r, x)
